# baseline (speedup 1.0000x reference)
import jax
import jax.numpy as jnp
from jax import lax
from jax.experimental import pallas as pl
from jax.experimental.pallas import tpu as pltpu

_N = 1024
_HID = 64
_M = 2
_SEM_HID = 128
_CH = 128
_NC = _N // _CH
_SLOPE = 0.2


def _fused_kernel(adj_hbm, feat_ref, aw_ref, gat_W_ref, gat_al_ref,
                  gat_ar_ref, gat_b_ref, gm_W_ref, gm_al_ref, gm_ar_ref,
                  gm_b_ref, sem_W1_ref, sem_b1_ref, sem_q_ref, ft_W_ref,
                  ft_b_ref, out_ref, abuf, sem):
    f32 = jnp.float32
    bf = jnp.bfloat16

    def copy(k):
        return pltpu.make_async_copy(
            adj_hbm.at[:, pl.ds(k * _CH, _CH), :], abuf.at[k], sem.at[k])

    for k in range(_NC):
        copy(k).start()

    feat = feat_ref[...]
    params = ((gat_W_ref[...], gat_al_ref[...].reshape(1, _HID),
               gat_ar_ref[...].reshape(1, _HID)),
              (gm_W_ref[0], gm_al_ref[0:1, :], gm_ar_ref[0:1, :]),
              (gm_W_ref[1], gm_al_ref[1:2, :], gm_ar_ref[1:2, :]))
    H, EL, ELM, ER, EC, R1, R2 = [], [], [], [], [], [], []
    for W, al, ar in params:
        h = jnp.dot(feat, W, preferred_element_type=f32)
        el = jnp.sum(h * al, axis=1, keepdims=True)
        elmax = jnp.max(el)
        u1 = jnp.exp(el - elmax)
        u2 = jnp.exp(_SLOPE * (el - elmax))
        H.append(h)
        EL.append(el)
        ELM.append(elmax)
        ER.append(lax.dot_general(ar, h, (((1,), (1,)), ((), ())),
                                  preferred_element_type=f32))
        EC.append(jnp.sum(h * ar, axis=1, keepdims=True))
        R1.append(jnp.concatenate([h * u1, u1], axis=1).astype(bf))
        R2.append(jnp.concatenate([h * u2, u2], axis=1).astype(bf))

    w = jax.nn.softmax(aw_ref[...].reshape(1, _M))
    dn = (((0,), (0,)), ((), ()))
    A1 = [None] * 3
    A2 = [None] * 3
    for k in range(_NC):
        copy(k).wait()
        a0f = abuf[k, 0].astype(f32)
        a1f = abuf[k, 1].astype(f32)
        mm = a0f * w[0:1, 0:1] + a1f * w[0:1, 1:2]
        cnt_m = jnp.where(mm != 0.0, 1.0, 0.0)
        lo, hi = k * _CH, (k + 1) * _CH
        for g, cnt in ((0, cnt_m), (1, a0f), (2, a1f)):
            x = EL[g][lo:hi, :] + ER[g]
            m1f = jnp.where(x >= 0.0, cnt, 0.0)
            m1 = m1f.astype(bf)
            m2 = (cnt - m1f).astype(bf)
            r1 = lax.dot_general(m1, R1[g][lo:hi, :], dn,
                                 preferred_element_type=f32)
            r2 = lax.dot_general(m2, R2[g][lo:hi, :], dn,
                                 preferred_element_type=f32)
            A1[g] = r1 if k == 0 else A1[g] + r1
            A2[g] = r2 if k == 0 else A2[g] + r2

    zs = []
    for g in range(3):
        t = ELM[g] + EC[g]
        c = jnp.where(t >= 0.0, t, _SLOPE * t)
        f1 = jnp.exp(t - c)
        f2 = jnp.exp(_SLOPE * t - c)
        xd = EL[g] + EC[g]
        ed = jnp.where(xd >= 0.0, xd, _SLOPE * xd)
        term = jnp.exp(ed - c)
        num = f1 * A1[g][:, :_HID] + f2 * A2[g][:, :_HID] + term * H[g]
        den = (f1 * A1[g][:, _HID:_HID + 1] + f2 * A2[g][:, _HID:_HID + 1]
               + term)
        zs.append(num / den)
    mg = jnp.tanh(zs[0] + gat_b_ref[...].reshape(1, _HID))
    m0 = jnp.tanh(zs[1] + gm_b_ref[0:1, :])
    m1_ = jnp.tanh(zs[2] + gm_b_ref[1:2, :])

    sem_W1 = sem_W1_ref[...]
    sem_b1 = sem_b1_ref[...].reshape(1, _SEM_HID)
    sem_q = sem_q_ref[...].reshape(1, _SEM_HID)

    def wp(xv):
        tt = jnp.tanh(jnp.dot(xv, sem_W1, preferred_element_type=f32)
                      + sem_b1)
        return jnp.sum(tt * sem_q)

    s0 = wp(mg) / _N
    s1 = wp(m0) / _N
    s2 = wp(m1_) / _N
    smax = jnp.maximum(jnp.maximum(s0, s1), s2)
    e0 = jnp.exp(s0 - smax)
    e1 = jnp.exp(s1 - smax)
    e2 = jnp.exp(s2 - smax)
    tot = e0 + e1 + e2
    semantic = (e0 / tot) * mg + (e1 / tot) * m0 + (e2 / tot) * m1_

    ft_W = ft_W_ref[...]
    fa = (jnp.dot(mg, ft_W[0:_HID, :], preferred_element_type=f32)
          + jnp.dot(semantic, ft_W[_HID:2 * _HID, :],
                    preferred_element_type=f32)
          + ft_b_ref[...].reshape(1, _HID))
    out_ref[...] = jnp.tanh(fa)


def kernel(adj_list, feat, attention_weights, gat_W, gat_al, gat_ar, gat_b,
           gm_W, gm_al, gm_ar, gm_b, sem_W1, sem_b1, sem_q, ft_W, ft_b):
    vmem = lambda: pl.BlockSpec(memory_space=pltpu.MemorySpace.VMEM)
    out = pl.pallas_call(
        _fused_kernel,
        in_specs=[pl.BlockSpec(memory_space=pl.ANY)] + [vmem()] * 15,
        out_specs=vmem(),
        out_shape=jax.ShapeDtypeStruct((_N, _HID), jnp.float32),
        scratch_shapes=[
            pltpu.VMEM((_NC, _M, _CH, _N), jnp.int32),
            pltpu.SemaphoreType.DMA((_NC,)),
        ],
    )(adj_list, feat, attention_weights, gat_W, gat_al, gat_ar, gat_b,
      gm_W, gm_al, gm_ar, gm_b, sem_W1, sem_b1, sem_q, ft_W, ft_b)
    return out


# speedup vs baseline: 1.0186x; 1.0186x over previous
import jax
import jax.numpy as jnp
from jax import lax
from jax.experimental import pallas as pl
from jax.experimental.pallas import tpu as pltpu

_N = 1024
_HID = 64
_M = 2
_SEM_HID = 128
_CH = 256
_NC = _N // _CH
_SLOPE = 0.2


def _fused_kernel(adj_hbm, feat_ref, aw_ref, gat_W_ref, gat_al_ref,
                  gat_ar_ref, gat_b_ref, gm_W_ref, gm_al_ref, gm_ar_ref,
                  gm_b_ref, sem_W1_ref, sem_b1_ref, sem_q_ref, ft_W_ref,
                  ft_b_ref, out_ref, abuf, sem):
    f32 = jnp.float32
    bf = jnp.bfloat16

    def copy(k):
        return pltpu.make_async_copy(
            adj_hbm.at[:, pl.ds(k * _CH, _CH), :], abuf.at[k], sem.at[k])

    for k in range(_NC):
        copy(k).start()

    feat = feat_ref[...]
    params = ((gat_W_ref[...], gat_al_ref[...].reshape(1, _HID),
               gat_ar_ref[...].reshape(1, _HID)),
              (gm_W_ref[0], gm_al_ref[0:1, :], gm_ar_ref[0:1, :]),
              (gm_W_ref[1], gm_al_ref[1:2, :], gm_ar_ref[1:2, :]))
    H, EL, ELM, ER, EC, R1, R2 = [], [], [], [], [], [], []
    for W, al, ar in params:
        h = jnp.dot(feat, W, preferred_element_type=f32)
        el = jnp.sum(h * al, axis=1, keepdims=True)
        elmax = jnp.max(el)
        u1 = jnp.exp(el - elmax)
        u2 = jnp.exp(_SLOPE * (el - elmax))
        H.append(h)
        EL.append(el)
        ELM.append(elmax)
        ER.append(lax.dot_general(ar, h, (((1,), (1,)), ((), ())),
                                  preferred_element_type=f32))
        EC.append(jnp.sum(h * ar, axis=1, keepdims=True))
        R1.append(jnp.concatenate([h * u1, u1], axis=1).astype(bf))
        R2.append(jnp.concatenate([h * u2, u2], axis=1).astype(bf))

    w = jax.nn.softmax(aw_ref[...].reshape(1, _M))
    dn = (((0,), (0,)), ((), ()))
    A1 = [None] * 3
    A2 = [None] * 3
    for k in range(_NC):
        copy(k).wait()
        a0f = abuf[k, 0].astype(f32)
        a1f = abuf[k, 1].astype(f32)
        mm = a0f * w[0:1, 0:1] + a1f * w[0:1, 1:2]
        cnt_m = jnp.where(mm != 0.0, 1.0, 0.0)
        lo, hi = k * _CH, (k + 1) * _CH
        for g, cnt in ((0, cnt_m), (1, a0f), (2, a1f)):
            x = EL[g][lo:hi, :] + ER[g]
            m1f = jnp.where(x >= 0.0, cnt, 0.0)
            m1 = m1f.astype(bf)
            m2 = (cnt - m1f).astype(bf)
            r1 = lax.dot_general(m1, R1[g][lo:hi, :], dn,
                                 preferred_element_type=f32)
            r2 = lax.dot_general(m2, R2[g][lo:hi, :], dn,
                                 preferred_element_type=f32)
            A1[g] = r1 if k == 0 else A1[g] + r1
            A2[g] = r2 if k == 0 else A2[g] + r2

    zs = []
    for g in range(3):
        t = ELM[g] + EC[g]
        c = jnp.where(t >= 0.0, t, _SLOPE * t)
        f1 = jnp.exp(t - c)
        f2 = jnp.exp(_SLOPE * t - c)
        xd = EL[g] + EC[g]
        ed = jnp.where(xd >= 0.0, xd, _SLOPE * xd)
        term = jnp.exp(ed - c)
        num = f1 * A1[g][:, :_HID] + f2 * A2[g][:, :_HID] + term * H[g]
        den = (f1 * A1[g][:, _HID:_HID + 1] + f2 * A2[g][:, _HID:_HID + 1]
               + term)
        zs.append(num / den)
    mg = jnp.tanh(zs[0] + gat_b_ref[...].reshape(1, _HID))
    m0 = jnp.tanh(zs[1] + gm_b_ref[0:1, :])
    m1_ = jnp.tanh(zs[2] + gm_b_ref[1:2, :])

    sem_W1 = sem_W1_ref[...]
    sem_b1 = sem_b1_ref[...].reshape(1, _SEM_HID)
    sem_q = sem_q_ref[...].reshape(1, _SEM_HID)

    def wp(xv):
        tt = jnp.tanh(jnp.dot(xv, sem_W1, preferred_element_type=f32)
                      + sem_b1)
        return jnp.sum(tt * sem_q)

    s0 = wp(mg) / _N
    s1 = wp(m0) / _N
    s2 = wp(m1_) / _N
    smax = jnp.maximum(jnp.maximum(s0, s1), s2)
    e0 = jnp.exp(s0 - smax)
    e1 = jnp.exp(s1 - smax)
    e2 = jnp.exp(s2 - smax)
    tot = e0 + e1 + e2
    semantic = (e0 / tot) * mg + (e1 / tot) * m0 + (e2 / tot) * m1_

    ft_W = ft_W_ref[...]
    fa = (jnp.dot(mg, ft_W[0:_HID, :], preferred_element_type=f32)
          + jnp.dot(semantic, ft_W[_HID:2 * _HID, :],
                    preferred_element_type=f32)
          + ft_b_ref[...].reshape(1, _HID))
    out_ref[...] = jnp.tanh(fa)


def kernel(adj_list, feat, attention_weights, gat_W, gat_al, gat_ar, gat_b,
           gm_W, gm_al, gm_ar, gm_b, sem_W1, sem_b1, sem_q, ft_W, ft_b):
    vmem = lambda: pl.BlockSpec(memory_space=pltpu.MemorySpace.VMEM)
    out = pl.pallas_call(
        _fused_kernel,
        in_specs=[pl.BlockSpec(memory_space=pl.ANY)] + [vmem()] * 15,
        out_specs=vmem(),
        out_shape=jax.ShapeDtypeStruct((_N, _HID), jnp.float32),
        scratch_shapes=[
            pltpu.VMEM((_NC, _M, _CH, _N), jnp.int32),
            pltpu.SemaphoreType.DMA((_NC,)),
        ],
    )(adj_list, feat, attention_weights, gat_W, gat_al, gat_ar, gat_b,
      gm_W, gm_al, gm_ar, gm_b, sem_W1, sem_b1, sem_q, ft_W, ft_b)
    return out


# speedup vs baseline: 1.0281x; 1.0093x over previous
import jax
import jax.numpy as jnp
from jax import lax
from jax.experimental import pallas as pl
from jax.experimental.pallas import tpu as pltpu

_N = 1024
_HID = 64
_M = 2
_SEM_HID = 128
_CH = 256
_NC = _N // _CH
_SLOPE = 0.2


def _fused_kernel(adj_hbm, feat_ref, aw_ref, gat_W_ref, gat_al_ref,
                  gat_ar_ref, gat_b_ref, gm_W_ref, gm_al_ref, gm_ar_ref,
                  gm_b_ref, sem_W1_ref, sem_b1_ref, sem_q_ref, ft_W_ref,
                  ft_b_ref, out_ref, abuf, sem):
    f32 = jnp.float32
    bf = jnp.bfloat16

    def copy(k):
        return pltpu.make_async_copy(
            adj_hbm.at[:, pl.ds(k * _CH, _CH), :], abuf.at[k], sem.at[k])

    for k in range(_NC):
        copy(k).start()

    feat = feat_ref[...]
    params = ((gat_W_ref[...], gat_al_ref[...].reshape(1, _HID),
               gat_ar_ref[...].reshape(1, _HID)),
              (gm_W_ref[0], gm_al_ref[0:1, :], gm_ar_ref[0:1, :]),
              (gm_W_ref[1], gm_al_ref[1:2, :], gm_ar_ref[1:2, :]))
    H, EL, ELM, ER, EC, R1, R2 = [], [], [], [], [], [], []
    for W, al, ar in params:
        h = jnp.dot(feat, W, preferred_element_type=f32)
        el = jnp.sum(h * al, axis=1, keepdims=True)
        elmax = jnp.max(el)
        u1 = jnp.exp(el - elmax)
        u2 = jnp.exp(_SLOPE * (el - elmax))
        H.append(h)
        EL.append(el)
        ELM.append(elmax)
        ER.append(lax.dot_general(ar, h, (((1,), (1,)), ((), ())),
                                  preferred_element_type=f32))
        EC.append(jnp.sum(h * ar, axis=1, keepdims=True))
        R1.append(jnp.concatenate([h * u1, u1], axis=1).astype(bf))
        R2.append(jnp.concatenate([h * u2, u2], axis=1).astype(bf))

    w = jax.nn.softmax(aw_ref[...].reshape(1, _M))
    sw0 = jnp.where(w[0, 0] != 0.0, 1.0, 0.0)
    sw1 = jnp.where(w[0, 1] != 0.0, 1.0, 0.0)
    dn = (((0,), (0,)), ((), ()))
    A1 = [None] * 3
    A2 = [None] * 3
    for k in range(_NC):
        copy(k).wait()
        a0f = abuf[k, 0].astype(f32)
        a1f = abuf[k, 1].astype(f32)
        cnt_m = jnp.maximum(a0f * sw0, a1f * sw1)
        lo, hi = k * _CH, (k + 1) * _CH
        for g, cnt in ((0, cnt_m), (1, a0f), (2, a1f)):
            x = EL[g][lo:hi, :] + ER[g]
            m1f = jnp.where(x >= 0.0, cnt, 0.0)
            m1 = m1f.astype(bf)
            m2 = (cnt - m1f).astype(bf)
            r1 = lax.dot_general(m1, R1[g][lo:hi, :], dn,
                                 preferred_element_type=f32)
            r2 = lax.dot_general(m2, R2[g][lo:hi, :], dn,
                                 preferred_element_type=f32)
            A1[g] = r1 if k == 0 else A1[g] + r1
            A2[g] = r2 if k == 0 else A2[g] + r2

    zs = []
    for g in range(3):
        t = ELM[g] + EC[g]
        c = jnp.where(t >= 0.0, t, _SLOPE * t)
        f1 = jnp.exp(t - c)
        f2 = jnp.exp(_SLOPE * t - c)
        xd = EL[g] + EC[g]
        ed = jnp.where(xd >= 0.0, xd, _SLOPE * xd)
        term = jnp.exp(ed - c)
        num = f1 * A1[g][:, :_HID] + f2 * A2[g][:, :_HID] + term * H[g]
        den = (f1 * A1[g][:, _HID:_HID + 1] + f2 * A2[g][:, _HID:_HID + 1]
               + term)
        zs.append(num / den)
    mg = jnp.tanh(zs[0] + gat_b_ref[...].reshape(1, _HID))
    m0 = jnp.tanh(zs[1] + gm_b_ref[0:1, :])
    m1_ = jnp.tanh(zs[2] + gm_b_ref[1:2, :])

    sem_W1 = sem_W1_ref[...]
    sem_b1 = sem_b1_ref[...].reshape(1, _SEM_HID)
    sem_q = sem_q_ref[...].reshape(1, _SEM_HID)

    def wp(xv):
        tt = jnp.tanh(jnp.dot(xv, sem_W1, preferred_element_type=f32)
                      + sem_b1)
        return jnp.sum(tt * sem_q)

    s0 = wp(mg) / _N
    s1 = wp(m0) / _N
    s2 = wp(m1_) / _N
    smax = jnp.maximum(jnp.maximum(s0, s1), s2)
    e0 = jnp.exp(s0 - smax)
    e1 = jnp.exp(s1 - smax)
    e2 = jnp.exp(s2 - smax)
    tot = e0 + e1 + e2
    semantic = (e0 / tot) * mg + (e1 / tot) * m0 + (e2 / tot) * m1_

    ft_W = ft_W_ref[...]
    fa = (jnp.dot(mg, ft_W[0:_HID, :], preferred_element_type=f32)
          + jnp.dot(semantic, ft_W[_HID:2 * _HID, :],
                    preferred_element_type=f32)
          + ft_b_ref[...].reshape(1, _HID))
    out_ref[...] = jnp.tanh(fa)


def kernel(adj_list, feat, attention_weights, gat_W, gat_al, gat_ar, gat_b,
           gm_W, gm_al, gm_ar, gm_b, sem_W1, sem_b1, sem_q, ft_W, ft_b):
    vmem = lambda: pl.BlockSpec(memory_space=pltpu.MemorySpace.VMEM)
    out = pl.pallas_call(
        _fused_kernel,
        in_specs=[pl.BlockSpec(memory_space=pl.ANY)] + [vmem()] * 15,
        out_specs=vmem(),
        out_shape=jax.ShapeDtypeStruct((_N, _HID), jnp.float32),
        scratch_shapes=[
            pltpu.VMEM((_NC, _M, _CH, _N), jnp.int32),
            pltpu.SemaphoreType.DMA((_NC,)),
        ],
    )(adj_list, feat, attention_weights, gat_W, gat_al, gat_ar, gat_b,
      gm_W, gm_al, gm_ar, gm_b, sem_W1, sem_b1, sem_q, ft_W, ft_b)
    return out
